# baseline (speedup 1.0000x reference)
import functools

import jax
import jax.numpy as jnp
from jax import lax
from jax.experimental import pallas as pl
from jax.experimental.pallas import tpu as pltpu
from jax.experimental.pallas import tpu_sc as plsc

_EMB_DIM = 32
_NUM_EMB = 8192
_TOK_BLK = 128
_CB_BLK = 512
_N_CB = _NUM_EMB // _CB_BLK
_NC = 2
_NS = 16
_NW = _NC * _NS


def _vq_body(z_ref, z2_ref, cb_ref, idx_ref, min_ref):
    j = pl.program_id(1)
    zt = z_ref[...]
    z2t = z2_ref[...]
    cbt = cb_ref[...]
    s = lax.dot_general(zt.astype(jnp.bfloat16), cbt.astype(jnp.bfloat16),
                        (((1,), (1,)), ((), ())),
                        preferred_element_type=jnp.float32)
    cb2 = jnp.sum(cbt * cbt, axis=1)
    d = (z2t + cb2[None, :]) - 2.0 * s
    lidx = jnp.argmin(d, axis=1)[:, None] + j * _CB_BLK
    lmin = jnp.min(d, axis=1, keepdims=True)
    prev_min = jnp.where(j == 0, jnp.inf, min_ref[...])
    prev_idx = jnp.where(j == 0, 0, idx_ref[...])
    better = lmin < prev_min
    min_ref[...] = jnp.where(better, lmin, prev_min)
    idx_ref[...] = jnp.where(better, lidx, prev_idx)


def _vq_argmin(zflat, z2, cb):
    n_tok = zflat.shape[0]
    grid = (n_tok // _TOK_BLK, _N_CB)
    idx = pl.pallas_call(
        _vq_body,
        grid=grid,
        in_specs=[
            pl.BlockSpec((_TOK_BLK, _EMB_DIM), lambda i, j: (i, 0)),
            pl.BlockSpec((_TOK_BLK, 1), lambda i, j: (i, 0)),
            pl.BlockSpec((_CB_BLK, _EMB_DIM), lambda i, j: (j, 0)),
        ],
        out_specs=pl.BlockSpec((_TOK_BLK, 1), lambda i, j: (i, 0)),
        out_shape=jax.ShapeDtypeStruct((n_tok, 1), jnp.int32),
        scratch_shapes=[pltpu.VMEM((_TOK_BLK, 1), jnp.float32)],
        compiler_params=pltpu.CompilerParams(
            dimension_semantics=("parallel", "arbitrary")),
    )(zflat, z2, cb)
    return idx[:, 0]


def _make_sc_gather(B, b_per_w):
    mesh = plsc.VectorSubcoreMesh(core_axis_name="c", subcore_axis_name="s")

    @functools.partial(
        pl.kernel, mesh=mesh,
        compiler_params=pltpu.CompilerParams(use_tc_tiling_on_sc=False),
        out_type=jax.ShapeDtypeStruct((B, _EMB_DIM), jnp.float32),
        scratch_types=[
            pltpu.VMEM((b_per_w,), jnp.int32),
            pltpu.VMEM((b_per_w, _EMB_DIM), jnp.float32),
            pltpu.SemaphoreType.DMA,
        ],
    )
    def k(table_hbm, idx_hbm, out_hbm, idx_v, rows_v, sem):
        wid = lax.axis_index("s") * _NC + lax.axis_index("c")
        base = wid * b_per_w
        pltpu.sync_copy(idx_hbm.at[pl.ds(base, b_per_w)], idx_v)
        pltpu.async_copy(table_hbm.at[idx_v], rows_v, sem).wait()
        pltpu.sync_copy(rows_v, out_hbm.at[pl.ds(base, b_per_w)])

    return k


def _sc_gather(cb, idx):
    n = idx.shape[0]
    pad = (-n) % (8 * _NW)
    idxp = jnp.concatenate([idx, jnp.zeros((pad,), jnp.int32)])
    B = n + pad
    out = _make_sc_gather(B, B // _NW)(cb, idxp)
    return out[:n]


def _conv(x, w, b, stride=1, pad=1):
    o = lax.conv_general_dilated(x, w, (stride, stride), [(pad, pad), (pad, pad)],
                                 dimension_numbers=('NCHW', 'OIHW', 'NCHW'))
    return o + b[None, :, None, None]


def _convT(x, w, b, stride=2, pad=1):
    k = w.shape[2]
    wf = jnp.flip(w, axis=(2, 3)).transpose(1, 0, 2, 3)
    p = k - 1 - pad
    o = lax.conv_general_dilated(x, wf, (1, 1), [(p, p), (p, p)],
                                 lhs_dilation=(stride, stride),
                                 dimension_numbers=('NCHW', 'OIHW', 'NCHW'))
    return o + b[None, :, None, None]


def _res(x, p, name):
    h = jax.nn.relu(_conv(x, p[name + '_w1'], p[name + '_b1']))
    h = _conv(h, p[name + '_w2'], p[name + '_b2'])
    return jax.nn.relu(x + h)


def _encode(x, p):
    x = jax.nn.relu(_conv(x, p['e_d0_w'], p['e_d0_b'], stride=2, pad=1))
    x = _res(x, p, 'e_s0_r0'); x = _res(x, p, 'e_s0_r1')
    x = jax.nn.relu(_conv(x, p['e_d1_w'], p['e_d1_b'], stride=2, pad=1))
    x = _res(x, p, 'e_s1_r0'); x = _res(x, p, 'e_s1_r1')
    return _conv(x, p['e_out_w'], p['e_out_b'])


def _decode(x, p):
    x = _conv(x, p['d_in_w'], p['d_in_b'])
    x = _res(x, p, 'd_s0_r0'); x = _res(x, p, 'd_s0_r1')
    x = jax.nn.relu(_convT(x, p['d_u0_w'], p['d_u0_b']))
    x = _res(x, p, 'd_s1_r0'); x = _res(x, p, 'd_s1_r1')
    return _convT(x, p['d_u1_w'], p['d_u1_b'])


def _quantize_pallas(args):
    flat, z2, cb = args
    idx = _vq_argmin(flat, z2, cb)
    return _sc_gather(cb, idx)


def _quantize_never(args):
    flat, _, _ = args
    return jnp.zeros_like(flat)


def kernel(x, params):
    z = _encode(x, params)
    cb = params['codebook']
    B, C, H, W = z.shape
    flat = z.transpose(0, 2, 3, 1).reshape(-1, C)
    z2 = (flat ** 2).sum(1, keepdims=True)
    pred = jnp.sum(x) > -1e30
    qflat = lax.cond(pred, _quantize_pallas, _quantize_never, (flat, z2, cb))
    q = qflat.reshape(B, H, W, C).transpose(0, 3, 1, 2)
    loss = 0.25 * jnp.mean((lax.stop_gradient(q) - z) ** 2)
    q_st = z + lax.stop_gradient(q - z)
    recon = _decode(q_st, params)
    return recon, loss
